# baseline (speedup 1.0000x reference)
import functools

import jax
import jax.numpy as jnp
from jax import lax
from jax.experimental import pallas as pl
from jax.experimental.pallas import tpu as pltpu
from jax.experimental.pallas import tpu_sc as plsc

N_USERS = 25000
N_ITEMS = 25000
N_NODES = N_USERS + N_ITEMS
D = 64
E = 800000
LAYERS = 4

NC = 2
NS = 16
HALF = 25088
NPAD = 2 * HALF
ACC_ROWS = 25216
ZONE = ACC_ROWS // NS
R_T = HALF // NS
RC = 112
NRC = R_T // RC
E_T = 50176
CE = 128
NCH = E_T // CE
BLK = 8
NB = NCH // BLK
E_PAD = NS * E_T
PAD_DST = 1 << 29
NCHP = 416
CPG = 2 * BLK

_params = pltpu.CompilerParams(needs_layout_passes=False, use_tc_tiling_on_sc=False)
_mesh = plsc.VectorSubcoreMesh(core_axis_name="c", subcore_axis_name="s")


def _rsqrt16(d):
    i = lax.bitcast_convert_type(d, jnp.int32)
    y = lax.bitcast_convert_type(jnp.int32(0x5F3759DF) - (i >> 1), jnp.float32)
    for _ in range(3):
        y = y * (1.5 - 0.5 * d * y * y)
    return y


@functools.partial(
    pl.kernel,
    out_type=(
        jax.ShapeDtypeStruct((NPAD,), jnp.float32),
        jax.ShapeDtypeStruct((NPAD, D), jnp.float32),
        jax.ShapeDtypeStruct((NC, NS, NCHP, CE), jnp.int32),
        jax.ShapeDtypeStruct((NC, NS, NCHP, CE), jnp.int32),
    ),
    mesh=_mesh,
    compiler_params=_params,
    scratch_types=dict(
        deg_sh=pltpu.VMEM_SHARED((ACC_ROWS,), jnp.float32),
        dst_blk=pltpu.VMEM((BLK, CE), jnp.int32),
        src_blk=pltpu.VMEM((BLK, CE), jnp.int32),
        dsti_blk=[pltpu.VMEM((BLK, CE), jnp.int32) for _ in range(2)],
        stage_s=pltpu.VMEM((2 * CE,), jnp.int32),
        stage_d=pltpu.VMEM((2 * CE,), jnp.int32),
        onesv=pltpu.VMEM((CE,), jnp.float32),
        zb1=pltpu.VMEM((ZONE,), jnp.float32),
        degv=pltpu.VMEM((RC,), jnp.float32),
        disv=pltpu.VMEM((RC,), jnp.float32),
        rb=pltpu.VMEM((RC, D), jnp.float32),
    ),
)
def _deg_dis_z0(dst_hbm, src_hbm, x0_hbm, zeros1d_hbm, ones_hbm,
                dis_out, z0_out, srcc_out, dstic_out,
                deg_sh, dst_blk, src_blk, dsti_blk, stage_s, stage_d,
                onesv, zb1, degv, disv, rb):
    c = lax.axis_index("c")
    s = lax.axis_index("s")
    dummy = HALF + s
    base = c * HALF

    pltpu.sync_copy(zeros1d_hbm, zb1)
    pltpu.sync_copy(zb1, deg_sh.at[pl.ds(s * ZONE, ZONE)])
    pltpu.sync_copy(ones_hbm, onesv)
    plsc.subcore_barrier()

    def _flush(off, nfl):
        pltpu.sync_copy(stage_s.at[pl.ds(0, CE)], srcc_out.at[c, s, nfl])
        pltpu.sync_copy(stage_d.at[pl.ds(0, CE)], dstic_out.at[c, s, nfl])
        for k in range(CE // 16):
            stage_s[pl.ds(k * 16, 16)] = stage_s[pl.ds(CE + k * 16, 16)]
            stage_d[pl.ds(k * 16, 16)] = stage_d[pl.ds(CE + k * 16, 16)]
        return off - CE, nfl + 1

    def _deg_block(blk, p, carry):
        off, nfl = carry
        pltpu.sync_copy(dst_hbm.at[s, pl.ds(blk * BLK, BLK)], dst_blk)
        pltpu.sync_copy(src_hbm.at[s, pl.ds(blk * BLK, BLK)], src_blk)
        for j in range(BLK):
            for k in range(CE // 16):
                dv = dst_blk[j, pl.ds(k * 16, 16)]
                ld = dv - base
                ok = (ld >= 0) & (ld < HALF)
                dsti_blk[p][j, pl.ds(k * 16, 16)] = jnp.where(ok, ld, dummy)
        for j in range(BLK):
            pltpu.sync_copy(onesv, deg_sh.at[dsti_blk[p].at[j]], add=True)
        for j in range(BLK):
            for k in range(CE // 16):
                dv = dst_blk[j, pl.ds(k * 16, 16)]
                ld = dv - base
                ok = (ld >= 0) & (ld < HALF)
                sv = src_blk[j, pl.ds(k * 16, 16)]
                plsc.store_compressed(stage_s.at[pl.ds(off, 16)], sv, mask=ok)
                plsc.store_compressed(stage_d.at[pl.ds(off, 16)], ld, mask=ok)
                off = off + jnp.sum(ok.astype(jnp.int32))
            off, nfl = lax.cond(off >= CE, _flush,
                                lambda o, n: (o, n), off, nfl)
        return off, nfl

    carry0 = (jnp.int32(0), jnp.int32(0))

    @pl.loop(0, NB - 1, step=2, init_carry=carry0)
    def _deg_blocks(g2, carry):
        carry = _deg_block(g2, 0, carry)
        return _deg_block(g2 + 1, 1, carry)

    off, nfl = _deg_block(NB - 1, 0, _deg_blocks)

    dum16 = jnp.full((16,), dummy, jnp.int32)
    zro16 = jnp.zeros((16,), jnp.int32)
    for k in range(CE // 16):
        stage_s[pl.ds(off + k * 16, 16)] = zro16
        stage_d[pl.ds(off + k * 16, 16)] = dum16
        stage_s[pl.ds(CE + k * 16, 16)] = zro16
        stage_d[pl.ds(CE + k * 16, 16)] = dum16
    @pl.loop(0, NCHP - nfl)
    def _pad_flush(q):
        pltpu.sync_copy(stage_s.at[pl.ds(0, CE)], srcc_out.at[c, s, nfl + q])
        pltpu.sync_copy(stage_d.at[pl.ds(0, CE)], dstic_out.at[c, s, nfl + q])
        for k in range(CE // 16):
            stage_s[pl.ds(k * 16, 16)] = stage_s[pl.ds(CE + k * 16, 16)]
            stage_d[pl.ds(k * 16, 16)] = stage_d[pl.ds(CE + k * 16, 16)]

    plsc.subcore_barrier()

    for j in range(NRC):
        lbase = s * R_T + j * RC
        gbase = c * HALF + lbase
        pltpu.sync_copy(deg_sh.at[pl.ds(lbase, RC)], degv)
        for k in range(RC // 16):
            dv = degv[pl.ds(k * 16, 16)]
            disv[pl.ds(k * 16, 16)] = jnp.where(dv > 0.0, _rsqrt16(dv), 0.0)
        pltpu.sync_copy(disv, dis_out.at[pl.ds(gbase, RC)])
        pltpu.sync_copy(x0_hbm.at[pl.ds(gbase, RC)], rb)

        @pl.loop(0, RC)
        def _scale_loop(r):
            b = plsc.load_gather(disv, [jnp.full((16,), r, jnp.int32)])
            for q in range(D // 16):
                rb[r, pl.ds(q * 16, 16)] = rb[r, pl.ds(q * 16, 16)] * b

        pltpu.sync_copy(rb, z0_out.at[pl.ds(gbase, RC)])


def _make_layer(s_scale, want_z):
    out_type = [jax.ShapeDtypeStruct((NPAD, D), jnp.float32)]
    if want_z:
        out_type.append(jax.ShapeDtypeStruct((NPAD, D), jnp.float32))

    @functools.partial(
        pl.kernel,
        out_type=tuple(out_type),
        mesh=_mesh,
        compiler_params=_params,
        scratch_types=dict(
            acc=pltpu.VMEM_SHARED((ACC_ROWS, D), jnp.float32),
            src_blk=[pltpu.VMEM((BLK, CE), jnp.int32) for _ in range(2)],
            dsti_blk=[pltpu.VMEM((BLK, CE), jnp.int32) for _ in range(2)],
            rows=[pltpu.VMEM((CE, D), jnp.float32) for _ in range(2)],
            gsem=[pltpu.SemaphoreType.DMA for _ in range(2)],
            ssem=[pltpu.SemaphoreType.DMA for _ in range(2)],
            disv=pltpu.VMEM((RC,), jnp.float32),
        ),
    )
    def _layer(srcc_hbm, dstic_hbm, dis_hbm, z_hbm, s_hbm, zeros2d_hbm,
               *outs, acc, src_blk, dsti_blk, rows, gsem, ssem, disv):
        s_out = outs[0]
        z_out = outs[1] if want_z else None
        rb = rows[1].at[pl.ds(0, RC)]
        sb = rows[0].at[pl.ds(0, RC)]

        c = lax.axis_index("c")
        s = lax.axis_index("s")

        pltpu.sync_copy(zeros2d_hbm, rows[0])
        for u in range(ZONE // CE):
            pltpu.sync_copy(rows[0], acc.at[pl.ds(s * ZONE + u * CE, CE)])
        rem = ZONE - (ZONE // CE) * CE
        pltpu.sync_copy(rows[0].at[pl.ds(0, rem)],
                        acc.at[pl.ds(s * ZONE + (ZONE // CE) * CE, rem)])
        plsc.subcore_barrier()

        def _gather(p, j, b):
            pltpu.async_copy(z_hbm.at[src_blk[p].at[j]], rows[b], gsem[b])

        def _wait_gather(b):
            pltpu.make_async_copy(z_hbm.at[src_blk[0].at[0]], rows[b], gsem[b]).wait()

        def _scatter(p, j, b):
            pltpu.async_copy(rows[b], acc.at[dsti_blk[p].at[j]], ssem[b], add=True)

        def _wait_scatter(b, guard=None):
            def _w():
                pltpu.make_async_copy(rows[b], acc.at[dsti_blk[0].at[0]],
                                      ssem[b]).wait()
            if guard is None:
                _w()
            else:
                pl.when(guard)(_w)

        def _edge_block(blk, p, wait_cond):
            pltpu.sync_copy(srcc_hbm.at[c, s, pl.ds(blk * BLK, BLK)],
                            src_blk[p])
            pltpu.sync_copy(dstic_hbm.at[c, s, pl.ds(blk * BLK, BLK)],
                            dsti_blk[p])
            _wait_scatter(0, wait_cond)
            _gather(p, 0, 0)
            for j in range(BLK):
                b = j % 2
                if j + 1 < BLK:
                    nb = (j + 1) % 2
                    _wait_scatter(nb, wait_cond if j == 0 else None)
                    _gather(p, j + 1, nb)
                _wait_gather(b)
                _scatter(p, j, b)

        @pl.loop(0, NCHP // CPG)
        def _edge_groups(g2):
            _edge_block(2 * g2, 0, g2 > 0)
            _edge_block(2 * g2 + 1, 1, None)

        _wait_scatter(0)
        _wait_scatter(1)
        plsc.subcore_barrier()

        for j in range(NRC):
            lbase = s * R_T + j * RC
            gbase = c * HALF + lbase
            pltpu.sync_copy(acc.at[pl.ds(lbase, RC)], rb)
            pltpu.sync_copy(dis_hbm.at[pl.ds(gbase, RC)], disv)
            pltpu.sync_copy(s_hbm.at[pl.ds(gbase, RC)], sb)

            @pl.loop(0, RC)
            def _scale_loop(r):
                bv = plsc.load_gather(disv, [jnp.full((16,), r, jnp.int32)])
                for q in range(D // 16):
                    sl = pl.ds(q * 16, 16)
                    x = rb[r, sl] * bv
                    sb[r, sl] = (sb[r, sl] + x) * s_scale
                    if want_z:
                        rb[r, sl] = x * bv

            pltpu.sync_copy(sb, s_out.at[pl.ds(gbase, RC)])
            if want_z:
                pltpu.sync_copy(rb, z_out.at[pl.ds(gbase, RC)])

    return _layer


_layer_mid = _make_layer(1.0, True)
_layer_last = _make_layer(1.0 / ((LAYERS + 1.0) ** 2), False)


def kernel(edge_index, emb_users, emb_items):
    src = edge_index[0]
    dst = edge_index[1]
    pad_e = E_PAD - E
    src_r = jnp.concatenate(
        [src, jnp.zeros((pad_e,), jnp.int32)]).reshape(NS, NCH, CE)
    dst_r = jnp.concatenate(
        [dst, jnp.full((pad_e,), PAD_DST, jnp.int32)]).reshape(NS, NCH, CE)
    x0 = jnp.concatenate(
        [emb_users, emb_items, jnp.zeros((NPAD - N_NODES, D), jnp.float32)], axis=0
    )
    zeros1d = jnp.zeros((ZONE,), jnp.float32)
    zeros2d = jnp.zeros((CE, D), jnp.float32)
    ones = jnp.ones((CE,), jnp.float32)

    dis, z, srcc, dstic = _deg_dis_z0(dst_r, src_r, x0, zeros1d, ones)
    S = x0
    for k in range(LAYERS):
        if k < LAYERS - 1:
            S, z = _layer_mid(srcc, dstic, dis, z, S, zeros2d)
        else:
            (S,) = _layer_last(srcc, dstic, dis, z, S, zeros2d)

    emb_users_final = S[:N_USERS]
    emb_items_final = S[N_USERS:N_NODES]
    return (emb_users_final, emb_users, emb_items_final, emb_items)


# speedup vs baseline: 19.9363x; 19.9363x over previous
import functools

import jax
import jax.numpy as jnp
from jax import lax
from jax.experimental import pallas as pl
from jax.experimental.pallas import tpu as pltpu
from jax.experimental.pallas import tpu_sc as plsc

N_USERS = 25000
N_ITEMS = 25000
N_NODES = N_USERS + N_ITEMS
D = 64
E = 800000
LAYERS = 4

NC = 2
NS = 16
HALF = 25088
NPAD = 2 * HALF
ACC_ROWS = 25216
ZONE = ACC_ROWS // NS
R_T = HALF // NS
RC = 112
NRC = R_T // RC
E_T = 50176
CE = 128
NCH = E_T // CE
BLK = 8
NB = NCH // BLK
E_PAD = NS * E_T
PAD_DST = 1 << 29
NCHP = 416
CPG = 2 * BLK

_params = pltpu.CompilerParams(needs_layout_passes=False, use_tc_tiling_on_sc=False)
_mesh = plsc.VectorSubcoreMesh(core_axis_name="c", subcore_axis_name="s")


def _rsqrt16(d):
    i = lax.bitcast_convert_type(d, jnp.int32)
    y = lax.bitcast_convert_type(jnp.int32(0x5F3759DF) - (i >> 1), jnp.float32)
    for _ in range(3):
        y = y * (1.5 - 0.5 * d * y * y)
    return y


@functools.partial(
    pl.kernel,
    out_type=(
        jax.ShapeDtypeStruct((NPAD,), jnp.float32),
        jax.ShapeDtypeStruct((NPAD, D), jnp.float32),
        jax.ShapeDtypeStruct((NC, NS, NCHP, CE), jnp.int32),
        jax.ShapeDtypeStruct((NC, NS, NCHP, CE), jnp.int32),
    ),
    mesh=_mesh,
    compiler_params=_params,
    scratch_types=dict(
        deg_sh=pltpu.VMEM_SHARED((ACC_ROWS,), jnp.float32),
        dst_blk=pltpu.VMEM((BLK, CE), jnp.int32),
        src_blk=pltpu.VMEM((BLK, CE), jnp.int32),
        dsti_blk=[pltpu.VMEM((BLK, CE), jnp.int32) for _ in range(2)],
        stage_s=pltpu.VMEM((2 * CE,), jnp.int32),
        stage_d=pltpu.VMEM((2 * CE,), jnp.int32),
        onesv=pltpu.VMEM((CE,), jnp.float32),
        zb1=pltpu.VMEM((ZONE,), jnp.float32),
        degv=pltpu.VMEM((RC,), jnp.float32),
        disv=pltpu.VMEM((RC,), jnp.float32),
        rb=pltpu.VMEM((RC, D), jnp.float32),
    ),
)
def _deg_dis_z0(dst_hbm, src_hbm, x0_hbm, zeros1d_hbm, ones_hbm,
                dis_out, z0_out, srcc_out, dstic_out,
                deg_sh, dst_blk, src_blk, dsti_blk, stage_s, stage_d,
                onesv, zb1, degv, disv, rb):
    c = lax.axis_index("c")
    s = lax.axis_index("s")
    dummy = HALF + s
    base = c * HALF

    pltpu.sync_copy(zeros1d_hbm, zb1)
    pltpu.sync_copy(zb1, deg_sh.at[pl.ds(s * ZONE, ZONE)])
    pltpu.sync_copy(ones_hbm, onesv)
    plsc.subcore_barrier()

    def _flush(off, nfl):
        pltpu.sync_copy(stage_s.at[pl.ds(0, CE)], srcc_out.at[c, s, nfl])
        pltpu.sync_copy(stage_d.at[pl.ds(0, CE)], dstic_out.at[c, s, nfl])
        for k in range(CE // 16):
            stage_s[pl.ds(k * 16, 16)] = stage_s[pl.ds(CE + k * 16, 16)]
            stage_d[pl.ds(k * 16, 16)] = stage_d[pl.ds(CE + k * 16, 16)]
        return off - CE, nfl + 1

    def _deg_block(blk, p, carry):
        off, nfl = carry
        pltpu.sync_copy(dst_hbm.at[s, pl.ds(blk * BLK, BLK)], dst_blk)
        pltpu.sync_copy(src_hbm.at[s, pl.ds(blk * BLK, BLK)], src_blk)
        for j in range(BLK):
            for k in range(CE // 16):
                dv = dst_blk[j, pl.ds(k * 16, 16)]
                ld = dv - base
                ok = (ld >= 0) & (ld < HALF)
                dsti_blk[p][j, pl.ds(k * 16, 16)] = jnp.where(ok, ld, dummy)
        for j in range(BLK):
            pltpu.sync_copy(onesv, deg_sh.at[dsti_blk[p].at[j]], add=True)
        for j in range(BLK):
            for k in range(CE // 16):
                dv = dst_blk[j, pl.ds(k * 16, 16)]
                ld = dv - base
                ok = (ld >= 0) & (ld < HALF)
                sv = src_blk[j, pl.ds(k * 16, 16)]
                plsc.store_compressed(stage_s.at[pl.ds(off, 16)], sv, mask=ok)
                plsc.store_compressed(stage_d.at[pl.ds(off, 16)], ld, mask=ok)
                off = off + jnp.sum(ok.astype(jnp.int32))
            off, nfl = lax.cond(off >= CE, _flush,
                                lambda o, n: (o, n), off, nfl)
        return off, nfl

    carry0 = (jnp.int32(0), jnp.int32(0))

    @pl.loop(0, NB - 1, step=2, init_carry=carry0)
    def _deg_blocks(g2, carry):
        carry = _deg_block(g2, 0, carry)
        return _deg_block(g2 + 1, 1, carry)

    off, nfl = _deg_block(NB - 1, 0, _deg_blocks)

    iota16 = jnp.arange(16, dtype=jnp.int32)
    for k in range(CE // 16):
        dums = k * 16 + iota16
        dumd = HALF + dums
        stage_s[pl.ds(off + k * 16, 16)] = dums
        stage_d[pl.ds(off + k * 16, 16)] = dumd
        stage_s[pl.ds(CE + k * 16, 16)] = dums
        stage_d[pl.ds(CE + k * 16, 16)] = dumd
    @pl.loop(0, NCHP - nfl)
    def _pad_flush(q):
        pltpu.sync_copy(stage_s.at[pl.ds(0, CE)], srcc_out.at[c, s, nfl + q])
        pltpu.sync_copy(stage_d.at[pl.ds(0, CE)], dstic_out.at[c, s, nfl + q])
        for k in range(CE // 16):
            stage_s[pl.ds(k * 16, 16)] = stage_s[pl.ds(CE + k * 16, 16)]
            stage_d[pl.ds(k * 16, 16)] = stage_d[pl.ds(CE + k * 16, 16)]

    plsc.subcore_barrier()

    for j in range(NRC):
        lbase = s * R_T + j * RC
        gbase = c * HALF + lbase
        pltpu.sync_copy(deg_sh.at[pl.ds(lbase, RC)], degv)
        for k in range(RC // 16):
            dv = degv[pl.ds(k * 16, 16)]
            disv[pl.ds(k * 16, 16)] = jnp.where(dv > 0.0, _rsqrt16(dv), 0.0)
        pltpu.sync_copy(disv, dis_out.at[pl.ds(gbase, RC)])
        pltpu.sync_copy(x0_hbm.at[pl.ds(gbase, RC)], rb)

        @pl.loop(0, RC)
        def _scale_loop(r):
            b = plsc.load_gather(disv, [jnp.full((16,), r, jnp.int32)])
            for q in range(D // 16):
                rb[r, pl.ds(q * 16, 16)] = rb[r, pl.ds(q * 16, 16)] * b

        pltpu.sync_copy(rb, z0_out.at[pl.ds(gbase, RC)])


def _make_layer(s_scale, want_z):
    out_type = [jax.ShapeDtypeStruct((NPAD, D), jnp.float32)]
    if want_z:
        out_type.append(jax.ShapeDtypeStruct((NPAD, D), jnp.float32))

    @functools.partial(
        pl.kernel,
        out_type=tuple(out_type),
        mesh=_mesh,
        compiler_params=_params,
        scratch_types=dict(
            acc=pltpu.VMEM_SHARED((ACC_ROWS, D), jnp.float32),
            src_blk=[pltpu.VMEM((BLK, CE), jnp.int32) for _ in range(2)],
            dsti_blk=[pltpu.VMEM((BLK, CE), jnp.int32) for _ in range(2)],
            rows=[pltpu.VMEM((CE, D), jnp.float32) for _ in range(2)],
            gsem=[pltpu.SemaphoreType.DMA for _ in range(2)],
            ssem=[pltpu.SemaphoreType.DMA for _ in range(2)],
            disv=pltpu.VMEM((RC,), jnp.float32),
        ),
    )
    def _layer(srcc_hbm, dstic_hbm, dis_hbm, z_hbm, s_hbm, zeros2d_hbm,
               *outs, acc, src_blk, dsti_blk, rows, gsem, ssem, disv):
        s_out = outs[0]
        z_out = outs[1] if want_z else None
        rb = rows[1].at[pl.ds(0, RC)]
        sb = rows[0].at[pl.ds(0, RC)]

        c = lax.axis_index("c")
        s = lax.axis_index("s")

        pltpu.sync_copy(zeros2d_hbm, rows[0])
        for u in range(ZONE // CE):
            pltpu.sync_copy(rows[0], acc.at[pl.ds(s * ZONE + u * CE, CE)])
        rem = ZONE - (ZONE // CE) * CE
        pltpu.sync_copy(rows[0].at[pl.ds(0, rem)],
                        acc.at[pl.ds(s * ZONE + (ZONE // CE) * CE, rem)])
        plsc.subcore_barrier()

        def _gather(p, j, b):
            pltpu.async_copy(z_hbm.at[src_blk[p].at[j]], rows[b], gsem[b])

        def _wait_gather(b):
            pltpu.make_async_copy(z_hbm.at[src_blk[0].at[0]], rows[b], gsem[b]).wait()

        def _scatter(p, j, b):
            pltpu.async_copy(rows[b], acc.at[dsti_blk[p].at[j]], ssem[b], add=True)

        def _wait_scatter(b, guard=None):
            def _w():
                pltpu.make_async_copy(rows[b], acc.at[dsti_blk[0].at[0]],
                                      ssem[b]).wait()
            if guard is None:
                _w()
            else:
                pl.when(guard)(_w)

        def _edge_block(blk, p, wait_cond):
            pltpu.sync_copy(srcc_hbm.at[c, s, pl.ds(blk * BLK, BLK)],
                            src_blk[p])
            pltpu.sync_copy(dstic_hbm.at[c, s, pl.ds(blk * BLK, BLK)],
                            dsti_blk[p])
            _wait_scatter(0, wait_cond)
            _gather(p, 0, 0)
            for j in range(BLK):
                b = j % 2
                if j + 1 < BLK:
                    nb = (j + 1) % 2
                    _wait_scatter(nb, wait_cond if j == 0 else None)
                    _gather(p, j + 1, nb)
                _wait_gather(b)
                _scatter(p, j, b)

        @pl.loop(0, NCHP // CPG)
        def _edge_groups(g2):
            _edge_block(2 * g2, 0, g2 > 0)
            _edge_block(2 * g2 + 1, 1, None)

        _wait_scatter(0)
        _wait_scatter(1)
        plsc.subcore_barrier()

        for j in range(NRC):
            lbase = s * R_T + j * RC
            gbase = c * HALF + lbase
            pltpu.sync_copy(acc.at[pl.ds(lbase, RC)], rb)
            pltpu.sync_copy(dis_hbm.at[pl.ds(gbase, RC)], disv)
            pltpu.sync_copy(s_hbm.at[pl.ds(gbase, RC)], sb)

            @pl.loop(0, RC)
            def _scale_loop(r):
                bv = plsc.load_gather(disv, [jnp.full((16,), r, jnp.int32)])
                for q in range(D // 16):
                    sl = pl.ds(q * 16, 16)
                    x = rb[r, sl] * bv
                    sb[r, sl] = (sb[r, sl] + x) * s_scale
                    if want_z:
                        rb[r, sl] = x * bv

            pltpu.sync_copy(sb, s_out.at[pl.ds(gbase, RC)])
            if want_z:
                pltpu.sync_copy(rb, z_out.at[pl.ds(gbase, RC)])

    return _layer


_layer_mid = _make_layer(1.0, True)
_layer_last = _make_layer(1.0 / ((LAYERS + 1.0) ** 2), False)


def kernel(edge_index, emb_users, emb_items):
    src = edge_index[0]
    dst = edge_index[1]
    pad_e = E_PAD - E
    src_r = jnp.concatenate(
        [src, jnp.zeros((pad_e,), jnp.int32)]).reshape(NS, NCH, CE)
    dst_r = jnp.concatenate(
        [dst, jnp.full((pad_e,), PAD_DST, jnp.int32)]).reshape(NS, NCH, CE)
    x0 = jnp.concatenate(
        [emb_users, emb_items, jnp.zeros((NPAD - N_NODES, D), jnp.float32)], axis=0
    )
    zeros1d = jnp.zeros((ZONE,), jnp.float32)
    zeros2d = jnp.zeros((CE, D), jnp.float32)
    ones = jnp.ones((CE,), jnp.float32)

    dis, z, srcc, dstic = _deg_dis_z0(dst_r, src_r, x0, zeros1d, ones)
    S = x0
    for k in range(LAYERS):
        if k < LAYERS - 1:
            S, z = _layer_mid(srcc, dstic, dis, z, S, zeros2d)
        else:
            (S,) = _layer_last(srcc, dstic, dis, z, S, zeros2d)

    emb_users_final = S[:N_USERS]
    emb_items_final = S[N_USERS:N_NODES]
    return (emb_users_final, emb_users, emb_items_final, emb_items)


# speedup vs baseline: 46.0788x; 2.3113x over previous
import functools

import jax
import jax.numpy as jnp
from jax import lax
from jax.experimental import pallas as pl
from jax.experimental.pallas import tpu as pltpu
from jax.experimental.pallas import tpu_sc as plsc

N_USERS = 25000
N_ITEMS = 25000
N_NODES = N_USERS + N_ITEMS
D = 64
E = 800000
LAYERS = 4

NC = 2
NS = 16
HALF = 25088
NPAD = 2 * HALF
ACC_ROWS = 25216
ZONE = ACC_ROWS // NS
R_T = HALF // NS
RC = 112
NRC = R_T // RC
E_T = 50176
CE = 128
NCH = E_T // CE
BLK = 8
NB = NCH // BLK
E_PAD = NS * E_T
PAD_DST = 1 << 29
NCHP = 416
CPG = 2 * BLK

_params = pltpu.CompilerParams(needs_layout_passes=False, use_tc_tiling_on_sc=False)
_mesh = plsc.VectorSubcoreMesh(core_axis_name="c", subcore_axis_name="s")


def _rsqrt16(d):
    i = lax.bitcast_convert_type(d, jnp.int32)
    y = lax.bitcast_convert_type(jnp.int32(0x5F3759DF) - (i >> 1), jnp.float32)
    for _ in range(3):
        y = y * (1.5 - 0.5 * d * y * y)
    return y


@functools.partial(
    pl.kernel,
    out_type=(
        jax.ShapeDtypeStruct((NPAD,), jnp.float32),
        jax.ShapeDtypeStruct((NPAD, D), jnp.float32),
        jax.ShapeDtypeStruct((NC, NS, NCHP, CE), jnp.int32),
        jax.ShapeDtypeStruct((NC, NS, NCHP, CE), jnp.int32),
        jax.ShapeDtypeStruct((NC, NS, 16), jnp.int32),
    ),
    mesh=_mesh,
    compiler_params=_params,
    scratch_types=dict(
        deg_sh=pltpu.VMEM_SHARED((ACC_ROWS,), jnp.float32),
        dst_blk=pltpu.VMEM((BLK, CE), jnp.int32),
        src_blk=pltpu.VMEM((BLK, CE), jnp.int32),
        dsti_blk=[pltpu.VMEM((BLK, CE), jnp.int32) for _ in range(2)],
        stage_s=pltpu.VMEM((2 * CE,), jnp.int32),
        stage_d=pltpu.VMEM((2 * CE,), jnp.int32),
        cntv=pltpu.VMEM((16,), jnp.int32),
        onesv=pltpu.VMEM((CE,), jnp.float32),
        zb1=pltpu.VMEM((ZONE,), jnp.float32),
        degv=pltpu.VMEM((RC,), jnp.float32),
        disv=pltpu.VMEM((RC,), jnp.float32),
        rb=pltpu.VMEM((RC, D), jnp.float32),
    ),
)
def _deg_dis_z0(dst_hbm, src_hbm, x0_hbm, zeros1d_hbm, ones_hbm,
                dis_out, z0_out, srcc_out, dstic_out, nb2_out,
                deg_sh, dst_blk, src_blk, dsti_blk, stage_s, stage_d, cntv,
                onesv, zb1, degv, disv, rb):
    c = lax.axis_index("c")
    s = lax.axis_index("s")
    dummy = HALF + s
    base = c * HALF

    pltpu.sync_copy(zeros1d_hbm, zb1)
    pltpu.sync_copy(zb1, deg_sh.at[pl.ds(s * ZONE, ZONE)])
    pltpu.sync_copy(ones_hbm, onesv)
    plsc.subcore_barrier()

    def _flush(off, nfl):
        pltpu.sync_copy(stage_s.at[pl.ds(0, CE)], srcc_out.at[c, s, nfl])
        pltpu.sync_copy(stage_d.at[pl.ds(0, CE)], dstic_out.at[c, s, nfl])
        for k in range(CE // 16):
            stage_s[pl.ds(k * 16, 16)] = stage_s[pl.ds(CE + k * 16, 16)]
            stage_d[pl.ds(k * 16, 16)] = stage_d[pl.ds(CE + k * 16, 16)]
        return off - CE, nfl + 1

    def _deg_block(blk, p, carry):
        off, nfl = carry
        pltpu.sync_copy(dst_hbm.at[s, pl.ds(blk * BLK, BLK)], dst_blk)
        pltpu.sync_copy(src_hbm.at[s, pl.ds(blk * BLK, BLK)], src_blk)
        for j in range(BLK):
            for k in range(CE // 16):
                dv = dst_blk[j, pl.ds(k * 16, 16)]
                ld = dv - base
                ok = (ld >= 0) & (ld < HALF)
                dsti_blk[p][j, pl.ds(k * 16, 16)] = jnp.where(ok, ld, dummy)
        for j in range(BLK):
            pltpu.sync_copy(onesv, deg_sh.at[dsti_blk[p].at[j]], add=True)
        for j in range(BLK):
            for k in range(CE // 16):
                dv = dst_blk[j, pl.ds(k * 16, 16)]
                ld = dv - base
                ok = (ld >= 0) & (ld < HALF)
                sv = src_blk[j, pl.ds(k * 16, 16)]
                plsc.store_compressed(stage_s.at[pl.ds(off, 16)], sv, mask=ok)
                plsc.store_compressed(stage_d.at[pl.ds(off, 16)], ld, mask=ok)
                off = off + jnp.sum(ok.astype(jnp.int32))
            off, nfl = lax.cond(off >= CE, _flush,
                                lambda o, n: (o, n), off, nfl)
        return off, nfl

    carry0 = (jnp.int32(0), jnp.int32(0))

    @pl.loop(0, NB - 1, step=2, init_carry=carry0)
    def _deg_blocks(g2, carry):
        carry = _deg_block(g2, 0, carry)
        return _deg_block(g2 + 1, 1, carry)

    off, nfl = _deg_block(NB - 1, 0, _deg_blocks)

    iota16 = jnp.arange(16, dtype=jnp.int32)
    for k in range(CE // 16):
        dums = k * 16 + iota16
        dumd = HALF + dums
        stage_s[pl.ds(off + k * 16, 16)] = dums
        stage_d[pl.ds(off + k * 16, 16)] = dumd
        stage_s[pl.ds(CE + k * 16, 16)] = dums
        stage_d[pl.ds(CE + k * 16, 16)] = dumd
    total = nfl + (off > 0).astype(jnp.int32)
    ngroups = (total + CPG - 1) // CPG
    target = ngroups * CPG

    @pl.loop(0, target - nfl)
    def _pad_flush(q):
        pltpu.sync_copy(stage_s.at[pl.ds(0, CE)], srcc_out.at[c, s, nfl + q])
        pltpu.sync_copy(stage_d.at[pl.ds(0, CE)], dstic_out.at[c, s, nfl + q])
        for k in range(CE // 16):
            stage_s[pl.ds(k * 16, 16)] = stage_s[pl.ds(CE + k * 16, 16)]
            stage_d[pl.ds(k * 16, 16)] = stage_d[pl.ds(CE + k * 16, 16)]

    cntv[pl.ds(0, 16)] = jnp.full((16,), ngroups, jnp.int32)
    pltpu.sync_copy(cntv, nb2_out.at[c, s])
    plsc.subcore_barrier()

    for j in range(NRC):
        lbase = s * R_T + j * RC
        gbase = c * HALF + lbase
        pltpu.sync_copy(deg_sh.at[pl.ds(lbase, RC)], degv)
        for k in range(RC // 16):
            dv = degv[pl.ds(k * 16, 16)]
            disv[pl.ds(k * 16, 16)] = jnp.where(dv > 0.0, _rsqrt16(dv), 0.0)
        pltpu.sync_copy(disv, dis_out.at[pl.ds(gbase, RC)])
        pltpu.sync_copy(x0_hbm.at[pl.ds(gbase, RC)], rb)

        @pl.loop(0, RC)
        def _scale_loop(r):
            b = plsc.load_gather(disv, [jnp.full((16,), r, jnp.int32)])
            for q in range(D // 16):
                rb[r, pl.ds(q * 16, 16)] = rb[r, pl.ds(q * 16, 16)] * b

        pltpu.sync_copy(rb, z0_out.at[pl.ds(gbase, RC)])


def _make_layer(s_scale, want_z):
    out_type = [jax.ShapeDtypeStruct((NPAD, D), jnp.float32)]
    if want_z:
        out_type.append(jax.ShapeDtypeStruct((NPAD, D), jnp.float32))

    @functools.partial(
        pl.kernel,
        out_type=tuple(out_type),
        mesh=_mesh,
        compiler_params=_params,
        scratch_types=dict(
            acc=pltpu.VMEM_SHARED((ACC_ROWS, D), jnp.float32),
            src_blk=[pltpu.VMEM((BLK, CE), jnp.int32) for _ in range(2)],
            dsti_blk=[pltpu.VMEM((BLK, CE), jnp.int32) for _ in range(2)],
            rows=[pltpu.VMEM((CE, D), jnp.float32) for _ in range(2)],
            gsem=[pltpu.SemaphoreType.DMA for _ in range(2)],
            ssem=[pltpu.SemaphoreType.DMA for _ in range(2)],
            disv=pltpu.VMEM((RC,), jnp.float32),
            cntv=pltpu.VMEM((16,), jnp.int32),
        ),
    )
    def _layer(srcc_hbm, dstic_hbm, nb2_hbm, dis_hbm, z_hbm, s_hbm,
               zeros2d_hbm,
               *outs, acc, src_blk, dsti_blk, rows, gsem, ssem, disv, cntv):
        s_out = outs[0]
        z_out = outs[1] if want_z else None
        rb = rows[1].at[pl.ds(0, RC)]
        sb = rows[0].at[pl.ds(0, RC)]

        c = lax.axis_index("c")
        s = lax.axis_index("s")

        pltpu.sync_copy(zeros2d_hbm, rows[0])
        for u in range(ZONE // CE):
            pltpu.sync_copy(rows[0], acc.at[pl.ds(s * ZONE + u * CE, CE)])
        rem = ZONE - (ZONE // CE) * CE
        pltpu.sync_copy(rows[0].at[pl.ds(0, rem)],
                        acc.at[pl.ds(s * ZONE + (ZONE // CE) * CE, rem)])
        plsc.subcore_barrier()

        def _gather(p, j, b):
            pltpu.async_copy(z_hbm.at[src_blk[p].at[j]], rows[b], gsem[b])

        def _wait_gather(b):
            pltpu.make_async_copy(z_hbm.at[src_blk[0].at[0]], rows[b], gsem[b]).wait()

        def _scatter(p, j, b):
            pltpu.async_copy(rows[b], acc.at[dsti_blk[p].at[j]], ssem[b], add=True)

        def _wait_scatter(b, guard=None):
            def _w():
                pltpu.make_async_copy(rows[b], acc.at[dsti_blk[0].at[0]],
                                      ssem[b]).wait()
            if guard is None:
                _w()
            else:
                pl.when(guard)(_w)

        pltpu.sync_copy(nb2_hbm.at[c, s], cntv)
        ngroups = jnp.max(cntv[pl.ds(0, 16)])
        def _edge_block(blk, p, wait_cond):
            pltpu.sync_copy(srcc_hbm.at[c, s, pl.ds(blk * BLK, BLK)],
                            src_blk[p])
            pltpu.sync_copy(dstic_hbm.at[c, s, pl.ds(blk * BLK, BLK)],
                            dsti_blk[p])
            _wait_scatter(0, wait_cond)
            _gather(p, 0, 0)
            for j in range(BLK):
                b = j % 2
                if j + 1 < BLK:
                    nb = (j + 1) % 2
                    _wait_scatter(nb, wait_cond if j == 0 else None)
                    _gather(p, j + 1, nb)
                _wait_gather(b)
                _scatter(p, j, b)

        @pl.loop(0, ngroups)
        def _edge_groups(g2):
            _edge_block(2 * g2, 0, g2 > 0)
            _edge_block(2 * g2 + 1, 1, None)

        @pl.when(ngroups > 0)
        def _drain():
            _wait_scatter(0)
            _wait_scatter(1)

        plsc.subcore_barrier()

        for j in range(NRC):
            lbase = s * R_T + j * RC
            gbase = c * HALF + lbase
            pltpu.sync_copy(acc.at[pl.ds(lbase, RC)], rb)
            pltpu.sync_copy(dis_hbm.at[pl.ds(gbase, RC)], disv)
            pltpu.sync_copy(s_hbm.at[pl.ds(gbase, RC)], sb)

            @pl.loop(0, RC)
            def _scale_loop(r):
                bv = plsc.load_gather(disv, [jnp.full((16,), r, jnp.int32)])
                for q in range(D // 16):
                    sl = pl.ds(q * 16, 16)
                    x = rb[r, sl] * bv
                    sb[r, sl] = (sb[r, sl] + x) * s_scale
                    if want_z:
                        rb[r, sl] = x * bv

            pltpu.sync_copy(sb, s_out.at[pl.ds(gbase, RC)])
            if want_z:
                pltpu.sync_copy(rb, z_out.at[pl.ds(gbase, RC)])

    return _layer


_layer_mid = _make_layer(1.0, True)
_layer_last = _make_layer(1.0 / ((LAYERS + 1.0) ** 2), False)


def kernel(edge_index, emb_users, emb_items):
    src = edge_index[0]
    dst = edge_index[1]
    pad_e = E_PAD - E
    src_r = jnp.concatenate(
        [src, jnp.zeros((pad_e,), jnp.int32)]).reshape(NS, NCH, CE)
    dst_r = jnp.concatenate(
        [dst, jnp.full((pad_e,), PAD_DST, jnp.int32)]).reshape(NS, NCH, CE)
    x0 = jnp.concatenate(
        [emb_users, emb_items, jnp.zeros((NPAD - N_NODES, D), jnp.float32)], axis=0
    )
    zeros1d = jnp.zeros((ZONE,), jnp.float32)
    zeros2d = jnp.zeros((CE, D), jnp.float32)
    ones = jnp.ones((CE,), jnp.float32)

    dis, z, srcc, dstic, nb2 = _deg_dis_z0(dst_r, src_r, x0, zeros1d, ones)
    S = x0
    for k in range(LAYERS):
        if k < LAYERS - 1:
            S, z = _layer_mid(srcc, dstic, nb2, dis, z, S, zeros2d)
        else:
            (S,) = _layer_last(srcc, dstic, nb2, dis, z, S, zeros2d)

    emb_users_final = S[:N_USERS]
    emb_items_final = S[N_USERS:N_NODES]
    return (emb_users_final, emb_users, emb_items_final, emb_items)
